# baseline (speedup 1.0000x reference)
import functools

import jax
import jax.numpy as jnp
from jax import lax
from jax.experimental import pallas as pl
from jax.experimental.pallas import tpu as pltpu
from jax.experimental.pallas import tpu_sc as plsc

B, C_U, H, W = 4, 256, 128, 128
R, D_GAT = 2048, 128
N = H * W
C_OUT = C_U + D_GAT

NC, NS, L = 2, 16, 16
NW = NC * NS
GPB = NW // B
CPT = D_GAT // GPB
PCH = 2048
NCHUNK = N // PCH

_sc_mesh = plsc.VectorSubcoreMesh(core_axis_name="c", subcore_axis_name="s")


@functools.partial(
    pl.kernel,
    mesh=_sc_mesh,
    compiler_params=pltpu.CompilerParams(
        use_tc_tiling_on_sc=False, needs_layout_passes=False
    ),
    out_type=jax.ShapeDtypeStruct((B, D_GAT, N), jnp.float32),
    scratch_types=[
        pltpu.VMEM((R, CPT), jnp.float32),
        pltpu.VMEM((PCH,), jnp.int32),
        pltpu.VMEM((CPT, PCH), jnp.float32),
    ],
)
def _sc_gather(fg_hbm, idx_hbm, out_hbm, tbl_v, idx_v, ob_v):
    wid = lax.axis_index("s") * NC + lax.axis_index("c")
    b = wid // GPB
    d0 = (wid % GPB) * CPT
    pltpu.sync_copy(fg_hbm.at[:, pl.ds(d0, CPT)], tbl_v)

    def chunk_body(ci, carry):
        pltpu.sync_copy(idx_hbm.at[b, pl.ds(ci * PCH, PCH)], idx_v)

        def grp(j, carry2):
            iv = idx_v[pl.ds(j * L, L)]
            for d in range(CPT):
                dvec = jnp.full((L,), d, jnp.int32)
                ob_v[d, pl.ds(j * L, L)] = plsc.load_gather(tbl_v, [iv, dvec])
            return carry2

        lax.fori_loop(0, PCH // L, grp, 0, unroll=False)
        pltpu.sync_copy(ob_v, out_hbm.at[b, pl.ds(d0, CPT), pl.ds(ci * PCH, PCH)])
        return carry

    lax.fori_loop(0, NCHUNK, chunk_body, 0, unroll=False)


def _tc_concat_body(fu_ref, g_ref, o_ref, sem_u, sem_g):
    cp_u = pltpu.make_async_copy(fu_ref, o_ref.at[:, 0:C_U, :], sem_u)
    cp_g = pltpu.make_async_copy(g_ref, o_ref.at[:, C_U:C_OUT, :], sem_g)
    cp_u.start()
    cp_g.start()
    cp_u.wait()
    cp_g.wait()


_tc_concat = pl.pallas_call(
    _tc_concat_body,
    in_specs=[
        pl.BlockSpec(memory_space=pl.ANY),
        pl.BlockSpec(memory_space=pl.ANY),
    ],
    out_specs=pl.BlockSpec(memory_space=pl.ANY),
    out_shape=jax.ShapeDtypeStruct((B, C_OUT, N), jnp.float32),
    scratch_shapes=[pltpu.SemaphoreType.DMA, pltpu.SemaphoreType.DMA],
)


def kernel(f_u, f_g, region_to_pixel_map):
    idx32 = region_to_pixel_map.reshape(B, N).astype(jnp.int32)
    g = _sc_gather(f_g, idx32)
    fu3 = f_u.reshape(B, C_U, N)
    out = _tc_concat(fu3, g)
    return out.reshape(B, C_OUT, H, W)


# speedup vs baseline: 8.2761x; 8.2761x over previous
import functools

import jax
import jax.numpy as jnp
from jax import lax
from jax.experimental import pallas as pl
from jax.experimental.pallas import tpu as pltpu
from jax.experimental.pallas import tpu_sc as plsc

B, C_U, H, W = 4, 256, 128, 128
R, D_GAT = 2048, 128
N = H * W
C_OUT = C_U + D_GAT

NC, NS, L = 2, 16, 16
NW = NC * NS
GPB = NW // B
CPT = D_GAT // GPB
PCH = 2048
NCHUNK = N // PCH

_sc_mesh = plsc.VectorSubcoreMesh(core_axis_name="c", subcore_axis_name="s")


@functools.partial(
    pl.kernel,
    mesh=_sc_mesh,
    compiler_params=pltpu.CompilerParams(
        use_tc_tiling_on_sc=False, needs_layout_passes=False
    ),
    out_type=jax.ShapeDtypeStruct((B, C_OUT, N), jnp.float32),
    scratch_types=[
        pltpu.VMEM((R, CPT), jnp.float32),
        pltpu.VMEM((PCH,), jnp.int32),
        pltpu.VMEM((CPT, PCH), jnp.float32),
    ],
)
def _sc_gather(fg_hbm, idx_hbm, out_hbm, tbl_v, idx_v, ob_v):
    wid = lax.axis_index("s") * NC + lax.axis_index("c")
    b = wid // GPB
    d0 = (wid % GPB) * CPT
    pltpu.sync_copy(fg_hbm.at[:, pl.ds(d0, CPT)], tbl_v)

    def chunk_body(ci, carry):
        pltpu.sync_copy(idx_hbm.at[b, pl.ds(ci * PCH, PCH)], idx_v)

        def grp(j, carry2):
            iv = idx_v[pl.ds(j * L, L)]
            for d in range(CPT):
                dvec = jnp.full((L,), d, jnp.int32)
                ob_v[d, pl.ds(j * L, L)] = plsc.load_gather(tbl_v, [iv, dvec])
            return carry2

        lax.fori_loop(0, PCH // L, grp, 0, unroll=False)
        pltpu.sync_copy(
            ob_v, out_hbm.at[b, pl.ds(C_U + d0, CPT), pl.ds(ci * PCH, PCH)]
        )
        return carry

    lax.fori_loop(0, NCHUNK, chunk_body, 0, unroll=False)


CB = 64


def _tc_fill_body(acc_ref, fu_ref, o_ref):
    del acc_ref
    o_ref[...] = fu_ref[...]


_tc_fill = pl.pallas_call(
    _tc_fill_body,
    grid=(B, C_U // CB),
    in_specs=[
        pl.BlockSpec(memory_space=pl.ANY),
        pl.BlockSpec((1, CB, N), lambda b, c: (b, c, 0)),
    ],
    out_specs=pl.BlockSpec((1, CB, N), lambda b, c: (b, c, 0)),
    out_shape=jax.ShapeDtypeStruct((B, C_OUT, N), jnp.float32),
    input_output_aliases={0: 0},
)


def kernel(f_u, f_g, region_to_pixel_map):
    idx32 = region_to_pixel_map.reshape(B, N).astype(jnp.int32)
    acc = _sc_gather(f_g, idx32)
    fu3 = f_u.reshape(B, C_U, N)
    out = _tc_fill(acc, fu3)
    return out.reshape(B, C_OUT, H, W)


# speedup vs baseline: 14.6618x; 1.7716x over previous
import functools

import jax
import jax.numpy as jnp
from jax import lax
from jax.experimental import pallas as pl
from jax.experimental.pallas import tpu as pltpu
from jax.experimental.pallas import tpu_sc as plsc

B, C_U, H, W = 4, 256, 128, 128
R, D_GAT = 2048, 128
N = H * W
C_OUT = C_U + D_GAT

NC, NS, L = 2, 16, 16
NW = NC * NS
GPB = NW // B
CPT = D_GAT // GPB
HCH = 16
PCH = HCH * W
NCHUNK = H // HCH
GPW = W // L

_sc_mesh = plsc.VectorSubcoreMesh(core_axis_name="c", subcore_axis_name="s")


@functools.partial(
    pl.kernel,
    mesh=_sc_mesh,
    compiler_params=pltpu.CompilerParams(
        use_tc_tiling_on_sc=False, needs_layout_passes=False
    ),
    out_type=jax.ShapeDtypeStruct((B, C_OUT, H, W), jnp.float32),
    scratch_types=[
        pltpu.VMEM((R, CPT), jnp.float32),
        pltpu.VMEM((HCH, W), jnp.int32),
        pltpu.VMEM((CPT, HCH, W), jnp.float32),
    ],
)
def _sc_gather(fg_hbm, idx_hbm, out_hbm, tbl_v, idx_v, ob_v):
    wid = lax.axis_index("s") * NC + lax.axis_index("c")
    b = wid // GPB
    d0 = (wid % GPB) * CPT
    pltpu.sync_copy(fg_hbm.at[:, pl.ds(d0, CPT)], tbl_v)

    def chunk_body(ci, carry):
        pltpu.sync_copy(idx_hbm.at[b, pl.ds(ci * HCH, HCH), :], idx_v)

        def grp(j, carry2):
            r = j // GPW
            c0 = (j % GPW) * L
            iv = idx_v[r, pl.ds(c0, L)]
            for d in range(CPT):
                dvec = jnp.full((L,), d, jnp.int32)
                ob_v[d, r, pl.ds(c0, L)] = plsc.load_gather(tbl_v, [iv, dvec])
            return carry2

        lax.fori_loop(0, HCH * GPW, grp, 0, unroll=False)
        pltpu.sync_copy(
            ob_v, out_hbm.at[b, pl.ds(C_U + d0, CPT), pl.ds(ci * HCH, HCH), :]
        )
        return carry

    lax.fori_loop(0, NCHUNK, chunk_body, 0, unroll=False)


CB = 64


def _tc_fill_body(acc_ref, fu_ref, o_ref):
    del acc_ref
    o_ref[...] = fu_ref[...]


_tc_fill = pl.pallas_call(
    _tc_fill_body,
    grid=(B, C_U // CB),
    in_specs=[
        pl.BlockSpec(memory_space=pl.ANY),
        pl.BlockSpec((1, CB, H, W), lambda b, c: (b, c, 0, 0)),
    ],
    out_specs=pl.BlockSpec((1, CB, H, W), lambda b, c: (b, c, 0, 0)),
    out_shape=jax.ShapeDtypeStruct((B, C_OUT, H, W), jnp.float32),
    input_output_aliases={0: 0},
)


def kernel(f_u, f_g, region_to_pixel_map):
    idx32 = region_to_pixel_map.astype(jnp.int32)
    acc = _sc_gather(f_g, idx32)
    return _tc_fill(acc, f_u)
